# baseline (speedup 1.0000x reference)
import functools

import jax
import jax.numpy as jnp
from jax.experimental import pallas as pl
from jax.experimental.pallas import tpu as pltpu

_EPS = 1e-5
_BI = 1024
_BH = 256
_LOG2E = 1.4426950408889634


def _cx_kernel(t_ref, i_ref, o_ref, tn_ref, iu_ref, kmax_ref, *, nb, p):
    n = pl.program_id(0)
    ib = pl.program_id(1)

    @pl.when(ib == 0)
    def _prologue():
        t_all = t_ref[...]
        tot = jnp.sum(jnp.sum(t_all, axis=0), axis=1, keepdims=True)
        mt = tot / (t_all.shape[0] * p)
        tc = t_ref[n] - mt
        tnorm = jnp.sqrt(jnp.sum(tc * tc, axis=0, keepdims=True))
        tn_ref[...] = tc / tnorm
        ic = i_ref[0] - mt
        inorm = jnp.sqrt(jnp.sum(ic * ic, axis=0, keepdims=True))
        iu_ref[...] = ic / inorm
        kmax_ref[...] = jnp.zeros_like(kmax_ref)

    tn = tn_ref[...]
    acc = kmax_ref[...]
    for j in range(_BI // _BH):
        cos = jax.lax.dot_general(
            iu_ref[:, pl.ds(ib * _BI + j * _BH, _BH)], tn,
            dimension_numbers=(((0,), (0,)), ((), ())),
            preferred_element_type=jnp.float32,
        )
        maxcos = jnp.max(cos, axis=1, keepdims=True)
        c2 = _LOG2E / (1.0 - maxcos + 2.0 * _EPS)
        c1 = _LOG2E - c2
        e = jnp.exp2(c1 + c2 * cos)
        s = jnp.sum(e, axis=1, keepdims=True)
        cs = (e * (1.0 / s)).reshape(_BH // 8, 8, e.shape[1])
        acc = jnp.maximum(acc, jnp.max(cs, axis=0))
    kmax_ref[...] = acc

    @pl.when(ib == nb - 1)
    def _epilogue():
        cs_mean = jnp.sum(jnp.max(kmax_ref[...], axis=0)) / p
        o_ref[...] = jnp.full(o_ref.shape, -jnp.log(cs_mean), jnp.float32)


def kernel(I_features, T_features):
    n, c, h, w = I_features.shape
    p = h * w
    i3 = I_features.reshape(n, c, p)
    t3 = T_features.reshape(n, c, p)
    nb = p // _BI

    out = pl.pallas_call(
        functools.partial(_cx_kernel, nb=nb, p=p),
        grid=(n, nb),
        in_specs=[
            pl.BlockSpec((n, c, p), lambda ni, bi: (0, 0, 0)),
            pl.BlockSpec((1, c, p), lambda ni, bi: (ni, 0, 0)),
        ],
        out_specs=pl.BlockSpec((1, 1, 128), lambda ni, bi: (ni, 0, 0)),
        out_shape=jax.ShapeDtypeStruct((n, 1, 128), jnp.float32),
        scratch_shapes=[
            pltpu.VMEM((c, p), jnp.float32),
            pltpu.VMEM((c, p), jnp.float32),
            pltpu.VMEM((8, p), jnp.float32),
        ],
        compiler_params=pltpu.CompilerParams(
            dimension_semantics=("parallel", "arbitrary"),
            vmem_limit_bytes=56 * 1024 * 1024,
        ),
        name="contextual_loss",
    )(t3, i3)
    return jnp.mean(out[:, 0, 0])
